# baseline (speedup 1.0000x reference)
import jax
import jax.numpy as jnp
from jax import lax
from jax.experimental import pallas as pl
from jax.experimental.pallas import tpu as pltpu
from jax.experimental.pallas import tpu_sc as plsc

VOCAB = 1000000
EMBED_DIM = 16
TAG_SIZE = 2
BATCH = 16384
HIST = 50

_INFO = plsc.get_sparse_core_info()
_NC = _INFO.num_cores
_NS = _INFO.num_subcores
_NW = _NC * _NS
_N = BATCH * HIST
_BPW = BATCH // _NW
_PER_W = _BPW * HIST
_CHUNK = 1280
_NCHUNK = _PER_W // _CHUNK
_SUB = 128
_NSUB = _CHUNK // _SUB
_GROUPS = _CHUNK // 16

_VBLK = 8192
_VGRID = -(-VOCAB // _VBLK)


_TPAD = 8


def _proj_body(embt_ref, w_ref, b_ref, out_ref):
    e = embt_ref[...]
    prod = jax.lax.dot_general(
        e, w_ref[...], (((0,), (1,)), ((), ())),
        preferred_element_type=jnp.float32)
    out_ref[...] = prod + b_ref[...]


@jax.jit
def _tc_proj(embt, wpad, bpad):
    return pl.pallas_call(
        _proj_body,
        grid=(_VGRID,),
        in_specs=[
            pl.BlockSpec((EMBED_DIM, _VBLK), lambda i: (0, i)),
            pl.BlockSpec((_TPAD, EMBED_DIM), lambda i: (0, 0)),
            pl.BlockSpec((1, _TPAD), lambda i: (0, 0)),
        ],
        out_specs=pl.BlockSpec((_VBLK, _TPAD), lambda i: (i, 0)),
        out_shape=jax.ShapeDtypeStruct((VOCAB, _TPAD), jnp.float32),
    )(embt, wpad, bpad)


def _body(x_hbm, proj_hbm, out_hbm, idx_v, rows0_v, rows1_v, out_v,
          sem0, sem1):
    wid = lax.axis_index("s") * _NC + lax.axis_index("c")
    base = wid * _PER_W

    lanes = lax.iota(jnp.int32, 16)

    pltpu.sync_copy(x_hbm.at[pl.ds(base, _PER_W)], idx_v)

    bufs = (rows0_v, rows1_v)
    sems = (sem0, sem1)

    def fire(c):
        handles = []
        for j in range(_NSUB):
            handles.append(pltpu.async_copy(
                proj_hbm.at[idx_v.at[pl.ds(c * _CHUNK + j * _SUB, _SUB)]],
                bufs[c % 2].at[pl.ds(j * _SUB, _SUB), :],
                sems[c % 2],
            ))
        return handles

    t0 = jnp.zeros((16,), dtype=jnp.int32)
    t1 = jnp.full((16,), 1, dtype=jnp.int32)

    pending = fire(0)
    for c in range(_NCHUNK):
        for h in pending:
            h.wait()
        if c + 1 < _NCHUNK:
            pending = fire(c + 1)
        rows = bufs[c % 2]

        def compute(g, carry):
            row_ids = g * 16 + lanes
            i_local = c * _CHUNK + row_ids
            b_off = i_local // HIST
            l_pos = i_local - b_off * HIST
            v0 = plsc.load_gather(rows, [row_ids, t0])
            v1 = plsc.load_gather(rows, [row_ids, t1])
            plsc.store_scatter(out_v, [l_pos, t0, b_off], v0)
            plsc.store_scatter(out_v, [l_pos, t1, b_off], v1)
            return carry

        lax.fori_loop(0, _GROUPS, compute, 0)

    pltpu.sync_copy(out_v, out_hbm.at[:, :, pl.ds(wid * _BPW, _BPW)])


@jax.jit
def _run(x_flat, proj):
    mesh = plsc.VectorSubcoreMesh(core_axis_name="c", subcore_axis_name="s")
    return pl.kernel(
        _body,
        out_type=jax.ShapeDtypeStruct((HIST, TAG_SIZE, BATCH), jnp.float32),
        mesh=mesh,
        scratch_types=[
            pltpu.VMEM((_PER_W,), jnp.int32),
            pltpu.VMEM((_CHUNK, _TPAD), jnp.float32),
            pltpu.VMEM((_CHUNK, _TPAD), jnp.float32),
            pltpu.VMEM((HIST, TAG_SIZE, _BPW), jnp.float32),
            pltpu.SemaphoreType.DMA,
            pltpu.SemaphoreType.DMA,
        ],
        compiler_params=pltpu.CompilerParams(
            needs_layout_passes=False, use_tc_tiling_on_sc=False),
    )(x_flat, proj)


def kernel(x, emb, W, b):
    x_flat = x.reshape(-1).astype(jnp.int32)
    embt = jnp.swapaxes(emb, 0, 1)
    wpad = jnp.pad(W.astype(jnp.float32), ((0, _TPAD - TAG_SIZE), (0, 0)))
    bpad = jnp.pad(b.astype(jnp.float32), (0, _TPAD - TAG_SIZE)).reshape(1, _TPAD)
    proj = _tc_proj(embt, wpad, bpad)
    out_t = _run(x_flat, proj)
    return jnp.transpose(out_t, (2, 0, 1))


# speedup vs baseline: 3.1277x; 3.1277x over previous
import jax
import jax.numpy as jnp
from jax import lax
from jax.experimental import pallas as pl
from jax.experimental.pallas import tpu as pltpu
from jax.experimental.pallas import tpu_sc as plsc

VOCAB = 1000000
EMBED_DIM = 16
TAG_SIZE = 2
BATCH = 16384
HIST = 50

_INFO = plsc.get_sparse_core_info()
_NC = _INFO.num_cores
_NS = _INFO.num_subcores
_NW = _NC * _NS
_N = BATCH * HIST
_BPW = BATCH // _NW
_PER_W = _BPW * HIST
_CHUNK = 1280
_NCHUNK = _PER_W // _CHUNK
_SUB = 128
_NSUB = _CHUNK // _SUB
_GROUPS = _CHUNK // 16

_VBLK = 8192
_VGRID = -(-VOCAB // _VBLK)


def _proj_body(embt_ref, w_ref, b_ref, out0_ref, out1_ref):
    e = embt_ref[...]
    prod = jax.lax.dot_general(
        w_ref[...], e, (((1,), (0,)), ((), ())),
        preferred_element_type=jnp.float32)
    prod = prod + b_ref[...]
    out0_ref[...] = prod[0]
    out1_ref[...] = prod[1]


@jax.jit
def _tc_proj(embt, W, b2d):
    return pl.pallas_call(
        _proj_body,
        grid=(_VGRID,),
        in_specs=[
            pl.BlockSpec((EMBED_DIM, _VBLK), lambda i: (0, i)),
            pl.BlockSpec((TAG_SIZE, EMBED_DIM), lambda i: (0, 0)),
            pl.BlockSpec((TAG_SIZE, 1), lambda i: (0, 0)),
        ],
        out_specs=[
            pl.BlockSpec((_VBLK,), lambda i: (i,)),
            pl.BlockSpec((_VBLK,), lambda i: (i,)),
        ],
        out_shape=[
            jax.ShapeDtypeStruct((VOCAB,), jnp.float32),
            jax.ShapeDtypeStruct((VOCAB,), jnp.float32),
        ],
    )(embt, W, b2d)


def _body(x_hbm, p0_hbm, p1_hbm, out_hbm, idx_v, t0a_v, t1a_v, t0b_v, t1b_v,
          out_v, sem0, sem1):
    wid = lax.axis_index("s") * _NC + lax.axis_index("c")
    base = wid * _PER_W

    lanes = lax.iota(jnp.int32, 16)

    pltpu.sync_copy(x_hbm.at[pl.ds(base, _PER_W)], idx_v)

    bufs = ((t0a_v, t1a_v), (t0b_v, t1b_v))
    sems = (sem0, sem1)

    def fire(c):
        t0buf, t1buf = bufs[c % 2]
        sem = sems[c % 2]
        handles = []
        for j in range(_NSUB):
            isl = idx_v.at[pl.ds(c * _CHUNK + j * _SUB, _SUB)]
            dsl = pl.ds(j * _SUB, _SUB)
            handles.append(pltpu.async_copy(p0_hbm.at[isl], t0buf.at[dsl], sem))
            handles.append(pltpu.async_copy(p1_hbm.at[isl], t1buf.at[dsl], sem))
        return handles

    t0i = jnp.zeros((16,), dtype=jnp.int32)
    t1i = jnp.full((16,), 1, dtype=jnp.int32)

    pending = fire(0)
    for c in range(_NCHUNK):
        for h in pending:
            h.wait()
        if c + 1 < _NCHUNK:
            pending = fire(c + 1)
        t0buf, t1buf = bufs[c % 2]

        def compute(g, carry):
            i_local = c * _CHUNK + g * 16 + lanes
            b_off = i_local // HIST
            l_pos = i_local - b_off * HIST
            v0 = t0buf[pl.ds(g * 16, 16)]
            v1 = t1buf[pl.ds(g * 16, 16)]
            plsc.store_scatter(out_v, [l_pos, t0i, b_off], v0)
            plsc.store_scatter(out_v, [l_pos, t1i, b_off], v1)
            return carry

        lax.fori_loop(0, _GROUPS, compute, 0)

    pltpu.sync_copy(out_v, out_hbm.at[:, :, pl.ds(wid * _BPW, _BPW)])


@jax.jit
def _run(x_flat, p0, p1):
    mesh = plsc.VectorSubcoreMesh(core_axis_name="c", subcore_axis_name="s")
    return pl.kernel(
        _body,
        out_type=jax.ShapeDtypeStruct((HIST, TAG_SIZE, BATCH), jnp.float32),
        mesh=mesh,
        scratch_types=[
            pltpu.VMEM((_PER_W,), jnp.int32),
            pltpu.VMEM((_CHUNK,), jnp.float32),
            pltpu.VMEM((_CHUNK,), jnp.float32),
            pltpu.VMEM((_CHUNK,), jnp.float32),
            pltpu.VMEM((_CHUNK,), jnp.float32),
            pltpu.VMEM((HIST, TAG_SIZE, _BPW), jnp.float32),
            pltpu.SemaphoreType.DMA,
            pltpu.SemaphoreType.DMA,
        ],
        compiler_params=pltpu.CompilerParams(
            needs_layout_passes=False, use_tc_tiling_on_sc=False),
    )(x_flat, p0, p1)


def kernel(x, emb, W, b):
    x_flat = x.reshape(-1).astype(jnp.int32)
    embt = jnp.swapaxes(emb, 0, 1)
    b2d = b.astype(jnp.float32).reshape(TAG_SIZE, 1)
    p0, p1 = _tc_proj(embt, W.astype(jnp.float32), b2d)
    out_t = _run(x_flat, p0, p1)
    return jnp.transpose(out_t, (2, 0, 1))


# speedup vs baseline: 4.0643x; 1.2995x over previous
import jax
import jax.numpy as jnp
from jax import lax
from jax.experimental import pallas as pl
from jax.experimental.pallas import tpu as pltpu
from jax.experimental.pallas import tpu_sc as plsc

VOCAB = 1000000
EMBED_DIM = 16
TAG_SIZE = 2
BATCH = 16384
HIST = 50

_INFO = plsc.get_sparse_core_info()
_NC = _INFO.num_cores
_NS = _INFO.num_subcores
_NW = _NC * _NS
_BPW = BATCH // _NW
_SUB = 128
_NSUB = _BPW // _SUB
_GROUPS = _BPW // 16

_VBLK = 65536
_VGRID = -(-VOCAB // _VBLK)


def _proj_body(embt_ref, w_ref, b_ref, out0_ref, out1_ref):
    e = embt_ref[...]
    prod = jax.lax.dot_general(
        w_ref[...], e, (((1,), (0,)), ((), ())),
        preferred_element_type=jnp.float32)
    prod = prod + b_ref[...]
    out0_ref[...] = prod[0]
    out1_ref[...] = prod[1]


@jax.jit
def _tc_proj(embt, W, b2d):
    return pl.pallas_call(
        _proj_body,
        grid=(_VGRID,),
        in_specs=[
            pl.BlockSpec((EMBED_DIM, _VBLK), lambda i: (0, i)),
            pl.BlockSpec((TAG_SIZE, EMBED_DIM), lambda i: (0, 0)),
            pl.BlockSpec((TAG_SIZE, 1), lambda i: (0, 0)),
        ],
        out_specs=[
            pl.BlockSpec((_VBLK,), lambda i: (i,)),
            pl.BlockSpec((_VBLK,), lambda i: (i,)),
        ],
        out_shape=[
            jax.ShapeDtypeStruct((VOCAB,), jnp.float32),
            jax.ShapeDtypeStruct((VOCAB,), jnp.float32),
        ],
    )(embt, W, b2d)


def _body(xt_hbm, p0_hbm, p1_hbm, out_hbm, idx_v, t0a_v, t1a_v, t0b_v, t1b_v,
          out_v, sem0, sem1):
    wid = lax.axis_index("s") * _NC + lax.axis_index("c")
    b0 = wid * _BPW

    pltpu.sync_copy(xt_hbm.at[:, pl.ds(b0, _BPW)], idx_v)

    bufs = ((t0a_v, t1a_v), (t0b_v, t1b_v))
    sems = (sem0, sem1)

    def fire(l):
        t0buf, t1buf = bufs[l % 2]
        sem = sems[l % 2]
        handles = []
        for j in range(_NSUB):
            isl = idx_v.at[l, pl.ds(j * _SUB, _SUB)]
            dsl = pl.ds(j * _SUB, _SUB)
            handles.append(pltpu.async_copy(p0_hbm.at[isl], t0buf.at[dsl], sem))
            handles.append(pltpu.async_copy(p1_hbm.at[isl], t1buf.at[dsl], sem))
        return handles

    pending = fire(0)
    for l in range(HIST):
        for h in pending:
            h.wait()
        if l + 1 < HIST:
            pending = fire(l + 1)
        t0buf, t1buf = bufs[l % 2]

        def move(g, carry):
            sl = pl.ds(g * 16, 16)
            out_v[l, 0, sl] = t0buf[sl]
            out_v[l, 1, sl] = t1buf[sl]
            return carry

        lax.fori_loop(0, _GROUPS, move, 0)

    pltpu.sync_copy(out_v, out_hbm.at[:, :, pl.ds(b0, _BPW)])


@jax.jit
def _run(xt, p0, p1):
    mesh = plsc.VectorSubcoreMesh(core_axis_name="c", subcore_axis_name="s")
    return pl.kernel(
        _body,
        out_type=jax.ShapeDtypeStruct((HIST, TAG_SIZE, BATCH), jnp.float32),
        mesh=mesh,
        scratch_types=[
            pltpu.VMEM((HIST, _BPW), jnp.int32),
            pltpu.VMEM((_BPW,), jnp.float32),
            pltpu.VMEM((_BPW,), jnp.float32),
            pltpu.VMEM((_BPW,), jnp.float32),
            pltpu.VMEM((_BPW,), jnp.float32),
            pltpu.VMEM((HIST, TAG_SIZE, _BPW), jnp.float32),
            pltpu.SemaphoreType.DMA,
            pltpu.SemaphoreType.DMA,
        ],
        compiler_params=pltpu.CompilerParams(
            needs_layout_passes=False, use_tc_tiling_on_sc=False),
    )(xt, p0, p1)


def kernel(x, emb, W, b):
    xt = jnp.swapaxes(x, 0, 1).astype(jnp.int32)
    embt = jnp.swapaxes(emb, 0, 1)
    b2d = b.astype(jnp.float32).reshape(TAG_SIZE, 1)
    p0, p1 = _tc_proj(embt, W.astype(jnp.float32), b2d)
    out_t = _run(xt, p0, p1)
    return jnp.transpose(out_t, (2, 0, 1))
